# Initial kernel scaffold; baseline (speedup 1.0000x reference)
import functools

import jax
import jax.numpy as jnp
from jax import lax
from jax.experimental import pallas as pl
from jax.experimental.pallas import tpu as pltpu
from jax.experimental.pallas import tpu_sc as plsc

N = 10000
E = 160000
D = 256
DH = 128
NT = 16
NC = 2
EPT = E // NT
CHUNK = 128
NCHUNK = (EPT + CHUNK - 1) // CHUNK
EPAD = NCHUNK * CHUNK
SROWS = 10240
ZR = 128
BM = 400
NBLK = N // BM

_mesh = plsc.VectorSubcoreMesh(core_axis_name="c", subcore_axis_name="s")


def _fill(ref, nrows, width, val):
    def body(i, carry):
        for k in range(width // 16):
            ref[i, pl.ds(k * 16, 16)] = jnp.full((16,), val, jnp.float32)
        return carry
    lax.fori_loop(0, nrows, body, 0)


@functools.partial(
    pl.kernel,
    out_type=jax.ShapeDtypeStruct((N, 16), jnp.float32),
    mesh=_mesh,
    scratch_types=[
        pltpu.VMEM((NCHUNK, CHUNK), jnp.int32),
        pltpu.VMEM((ZR, 16), jnp.float32),
        pltpu.VMEM((ZR, 16), jnp.float32),
        pltpu.VMEM_SHARED((SROWS, 16), jnp.float32),
    ],
)
def _deg(dst_hbm, out_hbm, dstv, ones_v, zer_v, shared):
    cid = lax.axis_index("c")
    sid = lax.axis_index("s")
    _fill(ones_v, ZR, 16, 1.0)
    _fill(zer_v, ZR, 16, 0.0)
    rows_per_tile = SROWS // NT
    for k in range(rows_per_tile // ZR):
        pltpu.sync_copy(zer_v, shared.at[pl.ds(sid * rows_per_tile + k * ZR, ZR)])
    plsc.subcore_barrier()
    pltpu.sync_copy(dst_hbm.at[sid], dstv)

    def body(j, carry):
        pltpu.sync_copy(ones_v, shared.at[dstv.at[j]], add=True)
        return carry
    lax.fori_loop(0, NCHUNK, body, 0)
    plsc.subcore_barrier()

    @pl.when(cid == 0)
    def _():
        out_per_tile = N // NT
        pltpu.sync_copy(shared.at[pl.ds(sid * out_per_tile, out_per_tile)],
                        out_hbm.at[pl.ds(sid * out_per_tile, out_per_tile)])


@functools.partial(
    pl.kernel,
    out_type=jax.ShapeDtypeStruct((NC * N, DH), jnp.float32),
    mesh=_mesh,
    scratch_types=[
        pltpu.VMEM((NCHUNK, CHUNK), jnp.int32),
        pltpu.VMEM((NCHUNK, CHUNK), jnp.int32),
        pltpu.VMEM((CHUNK, DH), jnp.float32),
        pltpu.VMEM((ZR, DH), jnp.float32),
        pltpu.VMEM_SHARED((SROWS, DH), jnp.float32),
    ],
)
def _agg(g_hbm, src_hbm, dst_hbm, out_hbm, srcv, dstv, rows, zbuf, shared):
    cid = lax.axis_index("c")
    sid = lax.axis_index("s")
    _fill(zbuf, ZR, DH, 0.0)
    rows_per_tile = SROWS // NT
    for k in range(rows_per_tile // ZR):
        pltpu.sync_copy(zbuf, shared.at[pl.ds(sid * rows_per_tile + k * ZR, ZR)])
    plsc.subcore_barrier()
    pltpu.sync_copy(src_hbm.at[cid, sid], srcv)
    pltpu.sync_copy(dst_hbm.at[sid], dstv)

    def body(j, carry):
        pltpu.sync_copy(g_hbm.at[srcv.at[j]], rows)
        pltpu.sync_copy(rows, shared.at[dstv.at[j]], add=True)
        return carry
    lax.fori_loop(0, NCHUNK, body, 0)
    plsc.subcore_barrier()

    out_per_tile = N // NT
    pltpu.sync_copy(shared.at[pl.ds(sid * out_per_tile, out_per_tile)],
                    out_hbm.at[pl.ds(cid * N + sid * out_per_tile, out_per_tile)])


def _dinv_from(degc_ref):
    deg = degc_ref[:, 0:1] + 1.0
    return lax.rsqrt(deg)


def _mm_scale_body(x_ref, w_ref, degc_ref, out_ref):
    dinv = _dinv_from(degc_ref)
    out_ref[...] = jnp.dot(x_ref[...], w_ref[...],
                           preferred_element_type=jnp.float32) * dinv


def _mm_scale(x, w, degcol):
    return pl.pallas_call(
        _mm_scale_body,
        grid=(NBLK, NC),
        in_specs=[
            pl.BlockSpec((BM, D), lambda i, c: (i, 0)),
            pl.BlockSpec((D, DH), lambda i, c: (0, c)),
            pl.BlockSpec((BM, 16), lambda i, c: (i, 0)),
        ],
        out_specs=pl.BlockSpec((BM, DH), lambda i, c: (c * NBLK + i, 0)),
        out_shape=jax.ShapeDtypeStruct((NC * N, DH), jnp.float32),
    )(x, w, degcol)


def _layer2_body(a0_ref, a1_ref, g0_ref, g1_ref, degc_ref, b_ref, w_ref, out_ref):
    dinv = _dinv_from(degc_ref)
    a = jnp.concatenate([a0_ref[...] + g0_ref[...],
                         a1_ref[...] + g1_ref[...]], axis=1)
    a = jnp.maximum(a * dinv + b_ref[...], 0.0)
    out_ref[...] = jnp.dot(a, w_ref[...],
                           preferred_element_type=jnp.float32) * dinv


def _layer2(acc, g, degcol, b1, w2):
    return pl.pallas_call(
        _layer2_body,
        grid=(NBLK, NC),
        in_specs=[
            pl.BlockSpec((BM, DH), lambda i, c: (i, 0)),
            pl.BlockSpec((BM, DH), lambda i, c: (NBLK + i, 0)),
            pl.BlockSpec((BM, DH), lambda i, c: (i, 0)),
            pl.BlockSpec((BM, DH), lambda i, c: (NBLK + i, 0)),
            pl.BlockSpec((BM, 16), lambda i, c: (i, 0)),
            pl.BlockSpec((1, D), lambda i, c: (0, 0)),
            pl.BlockSpec((D, DH), lambda i, c: (0, c)),
        ],
        out_specs=pl.BlockSpec((BM, DH), lambda i, c: (c * NBLK + i, 0)),
        out_shape=jax.ShapeDtypeStruct((NC * N, DH), jnp.float32),
    )(acc, acc, g, g, degcol, b1, w2)


def _final_body(a0_ref, a1_ref, g0_ref, g1_ref, degc_ref, b_ref, out_ref):
    dinv = _dinv_from(degc_ref)
    a = jnp.concatenate([a0_ref[...] + g0_ref[...],
                         a1_ref[...] + g1_ref[...]], axis=1)
    out_ref[...] = a * dinv + b_ref[...]


def _final(acc, g, degcol, b2):
    return pl.pallas_call(
        _final_body,
        grid=(NBLK,),
        in_specs=[
            pl.BlockSpec((BM, DH), lambda i: (i, 0)),
            pl.BlockSpec((BM, DH), lambda i: (NBLK + i, 0)),
            pl.BlockSpec((BM, DH), lambda i: (i, 0)),
            pl.BlockSpec((BM, DH), lambda i: (NBLK + i, 0)),
            pl.BlockSpec((BM, 16), lambda i: (i, 0)),
            pl.BlockSpec((1, D), lambda i: (0, 0)),
        ],
        out_specs=pl.BlockSpec((BM, D), lambda i: (i, 0)),
        out_shape=jax.ShapeDtypeStruct((N, D), jnp.float32),
    )(acc, acc, g, g, degcol, b2)


def kernel(edges, x, W1, b1, W2, b2):
    src = edges[:, 0]
    dst = edges[:, 1]
    pad = EPAD - EPT
    srcp = jnp.pad(src.reshape(NT, EPT), ((0, 0), (0, pad)))
    dstp = jnp.pad(dst.reshape(NT, EPT), ((0, 0), (0, pad)),
                   constant_values=N)
    src2 = jnp.stack([srcp, srcp + N]).reshape(NC, NT, NCHUNK, CHUNK)
    dst3 = dstp.reshape(NT, NCHUNK, CHUNK)

    degcol = _deg(dst3)
    b1r = b1.reshape(1, D)
    b2r = b2.reshape(1, D)

    g1 = _mm_scale(x, W1, degcol)
    acc1 = _agg(g1, src2, dst3)
    g2 = _layer2(acc1, g1, degcol, b1r, W2)
    acc2 = _agg(g2, src2, dst3)
    return _final(acc2, g2, degcol, b2r)


# speedup vs baseline: 9.5902x; 9.5902x over previous
import functools

import jax
import jax.numpy as jnp
from jax import lax
from jax.experimental import pallas as pl
from jax.experimental.pallas import tpu as pltpu
from jax.experimental.pallas import tpu_sc as plsc

N = 10000
E = 160000
D = 256
DH = 128
NT = 16
NC = 2
EPT = E // NT
CHUNK = 128
NCHUNK = (EPT + CHUNK - 1) // CHUNK
EPAD = NCHUNK * CHUNK
SROWS = 10240
ZR = 128
BM = 400
NBLK = N // BM

@functools.lru_cache(maxsize=None)
def _mesh():
    return plsc.VectorSubcoreMesh(core_axis_name="c", subcore_axis_name="s",
                                  num_cores=NC, num_subcores=NT)


def _fill(ref, nrows, width, val):
    def body(i, carry):
        for k in range(width // 16):
            ref[i, pl.ds(k * 16, 16)] = jnp.full((16,), val, jnp.float32)
        return carry
    lax.fori_loop(0, nrows, body, 0)


@functools.lru_cache(maxsize=None)
def _make_deg():
    return pl.kernel(
        _deg_body,
        out_type=jax.ShapeDtypeStruct((SROWS, 16), jnp.float32),
        mesh=_mesh(),
        scratch_types=[
            pltpu.VMEM((NCHUNK, CHUNK), jnp.int32),
            pltpu.VMEM((ZR, 16), jnp.float32),
            pltpu.VMEM((ZR, 16), jnp.float32),
            pltpu.VMEM_SHARED((SROWS, 16), jnp.float32),
        ],
    )


def _deg_body(dst_hbm, out_hbm, dstv, ones_v, zer_v, shared):
    cid = lax.axis_index("c")
    sid = lax.axis_index("s")
    _fill(ones_v, ZR, 16, 1.0)
    _fill(zer_v, ZR, 16, 0.0)
    rows_per_tile = SROWS // NT
    for k in range(rows_per_tile // ZR):
        pltpu.sync_copy(zer_v, shared.at[pl.ds(sid * rows_per_tile + k * ZR, ZR)])
    plsc.subcore_barrier()
    pltpu.sync_copy(dst_hbm.at[sid], dstv)

    def body(j, carry):
        pltpu.sync_copy(ones_v, shared.at[dstv.at[j]], add=True)
        return carry
    lax.fori_loop(0, NCHUNK, body, 0)
    plsc.subcore_barrier()

    @pl.when(cid == 0)
    def _():
        out_per_tile = SROWS // NT
        pltpu.sync_copy(shared.at[pl.ds(sid * out_per_tile, out_per_tile)],
                        out_hbm.at[pl.ds(sid * out_per_tile, out_per_tile)])


@functools.lru_cache(maxsize=None)
def _make_agg():
    return pl.kernel(
        _agg_body,
        out_type=jax.ShapeDtypeStruct((NC, SROWS, DH), jnp.float32),
        mesh=_mesh(),
        scratch_types=[
            pltpu.VMEM((NCHUNK, CHUNK), jnp.int32),
            pltpu.VMEM((NCHUNK, CHUNK), jnp.int32),
            pltpu.VMEM((CHUNK, DH), jnp.float32),
            pltpu.VMEM_SHARED((SROWS, DH), jnp.float32),
        ],
    )


def _agg_body(g_hbm, src_hbm, dst_hbm, out_hbm, srcv, dstv, rows, shared):
    cid = lax.axis_index("c")
    sid = lax.axis_index("s")
    _fill(rows, ZR, DH, 0.0)
    rows_per_tile = SROWS // NT
    for k in range(rows_per_tile // ZR):
        pltpu.sync_copy(rows, shared.at[pl.ds(sid * rows_per_tile + k * ZR, ZR)])
    plsc.subcore_barrier()
    pltpu.sync_copy(src_hbm.at[cid, sid], srcv)
    pltpu.sync_copy(dst_hbm.at[sid], dstv)

    def body(j, carry):
        pltpu.sync_copy(g_hbm.at[srcv.at[j]], rows)
        pltpu.sync_copy(rows, shared.at[dstv.at[j]], add=True)
        return carry
    lax.fori_loop(0, NCHUNK, body, 0)
    plsc.subcore_barrier()

    out_per_tile = SROWS // NT
    pltpu.sync_copy(shared.at[pl.ds(sid * out_per_tile, out_per_tile)],
                    out_hbm.at[cid, pl.ds(sid * out_per_tile, out_per_tile)])


def _dinv_from(degc_ref):
    deg = degc_ref[:, 0:1] + 1.0
    return lax.rsqrt(deg)


def _mm_scale_body(x_ref, w_ref, degc_ref, out_ref):
    dinv = _dinv_from(degc_ref)
    out_ref[...] = jnp.dot(x_ref[...], w_ref[...],
                           preferred_element_type=jnp.float32) * dinv


def _mm_scale(x, w, degcol):
    return pl.pallas_call(
        _mm_scale_body,
        grid=(NBLK, NC),
        in_specs=[
            pl.BlockSpec((BM, D), lambda i, c: (i, 0)),
            pl.BlockSpec((D, DH), lambda i, c: (0, c)),
            pl.BlockSpec((BM, 16), lambda i, c: (i, 0)),
        ],
        out_specs=pl.BlockSpec((BM, DH), lambda i, c: (c * NBLK + i, 0)),
        out_shape=jax.ShapeDtypeStruct((NC * N, DH), jnp.float32),
    )(x, w, degcol)


def _layer2_body(a0_ref, a1_ref, g0_ref, g1_ref, degc_ref, b_ref, w_ref, out_ref):
    dinv = _dinv_from(degc_ref)
    a = jnp.concatenate([a0_ref[0] + g0_ref[...],
                         a1_ref[0] + g1_ref[...]], axis=1)
    a = jnp.maximum(a * dinv + b_ref[...], 0.0)
    out_ref[...] = jnp.dot(a, w_ref[...],
                           preferred_element_type=jnp.float32) * dinv


def _layer2(acc, g, degcol, b1, w2):
    return pl.pallas_call(
        _layer2_body,
        grid=(NBLK, NC),
        in_specs=[
            pl.BlockSpec((1, BM, DH), lambda i, c: (0, i, 0)),
            pl.BlockSpec((1, BM, DH), lambda i, c: (1, i, 0)),
            pl.BlockSpec((BM, DH), lambda i, c: (i, 0)),
            pl.BlockSpec((BM, DH), lambda i, c: (NBLK + i, 0)),
            pl.BlockSpec((BM, 16), lambda i, c: (i, 0)),
            pl.BlockSpec((1, D), lambda i, c: (0, 0)),
            pl.BlockSpec((D, DH), lambda i, c: (0, c)),
        ],
        out_specs=pl.BlockSpec((BM, DH), lambda i, c: (c * NBLK + i, 0)),
        out_shape=jax.ShapeDtypeStruct((NC * N, DH), jnp.float32),
    )(acc, acc, g, g, degcol, b1, w2)


def _final_body(a0_ref, a1_ref, g0_ref, g1_ref, degc_ref, b_ref, out_ref):
    dinv = _dinv_from(degc_ref)
    a = jnp.concatenate([a0_ref[0] + g0_ref[...],
                         a1_ref[0] + g1_ref[...]], axis=1)
    out_ref[...] = a * dinv + b_ref[...]


def _final(acc, g, degcol, b2):
    return pl.pallas_call(
        _final_body,
        grid=(NBLK,),
        in_specs=[
            pl.BlockSpec((1, BM, DH), lambda i: (0, i, 0)),
            pl.BlockSpec((1, BM, DH), lambda i: (1, i, 0)),
            pl.BlockSpec((BM, DH), lambda i: (i, 0)),
            pl.BlockSpec((BM, DH), lambda i: (NBLK + i, 0)),
            pl.BlockSpec((BM, 16), lambda i: (i, 0)),
            pl.BlockSpec((1, D), lambda i: (0, 0)),
        ],
        out_specs=pl.BlockSpec((BM, D), lambda i: (i, 0)),
        out_shape=jax.ShapeDtypeStruct((N, D), jnp.float32),
    )(acc, acc, g, g, degcol, b2)


def kernel(edges, x, W1, b1, W2, b2):
    src = edges[:, 0]
    dst = edges[:, 1]
    pad = EPAD - EPT
    srcp = jnp.pad(src.reshape(NT, EPT), ((0, 0), (0, pad)))
    dstp = jnp.pad(dst.reshape(NT, EPT), ((0, 0), (0, pad)),
                   constant_values=N)
    src2 = jnp.stack([srcp, srcp + N]).reshape(NC, NT, NCHUNK, CHUNK)
    dst3 = dstp.reshape(NT, NCHUNK, CHUNK)

    degcol = _make_deg()(dst3)
    b1r = b1.reshape(1, D)
    b2r = b2.reshape(1, D)

    agg = _make_agg()
    g1 = _mm_scale(x, W1, degcol)
    acc1 = agg(g1, src2, dst3)
    g2 = _layer2(acc1, g1, degcol, b1r, W2)
    acc2 = agg(g2, src2, dst3)
    return _final(acc2, g2, degcol, b2r)
